# baseline (speedup 1.0000x reference)
import functools

import jax
import jax.numpy as jnp
from jax import lax
from jax.experimental import pallas as pl
from jax.experimental.pallas import tpu as pltpu
from jax.experimental.pallas import tpu_sc as plsc

_NC = 2
_NS = 16


@functools.lru_cache(maxsize=None)
def _make_gather(n_idx, row_w, n_chunks):
    nw = _NC * _NS
    per_w = n_idx // nw
    ch = per_w // n_chunks
    assert per_w % n_chunks == 0 and ch % 8 == 0

    mesh = plsc.VectorSubcoreMesh(
        core_axis_name="c", subcore_axis_name="s",
        num_cores=_NC, num_subcores=_NS)

    @functools.partial(
        pl.kernel,
        out_type=jax.ShapeDtypeStruct((n_idx, row_w), jnp.float32),
        mesh=mesh,
        scratch_types=[
            pltpu.VMEM((2, ch), jnp.int32),
            pltpu.VMEM((2, ch, row_w), jnp.float32),
            pltpu.SemaphoreType.DMA((2,)),
            pltpu.SemaphoreType.DMA((2,)),
        ],
        compiler_params=pltpu.CompilerParams(use_tc_tiling_on_sc=False),
    )
    def gather_k(tab_hbm, idx_hbm, out_hbm, idx_v, rows_v, gsem, wsem):
        wid = lax.axis_index("s") * _NC + lax.axis_index("c")
        base = wid * per_w

        def start_gather(c, b):
            pltpu.sync_copy(idx_hbm.at[pl.ds(base + c * ch, ch)], idx_v.at[b])
            return pltpu.async_copy(
                tab_hbm.at[idx_v.at[b]], rows_v.at[b], gsem.at[b])

        gathers = {0: start_gather(0, 0)}
        writes = {}
        for c in range(n_chunks):
            b = c % 2
            if c + 1 < n_chunks:
                if c - 1 >= 0:
                    writes[c - 1].wait()
                gathers[c + 1] = start_gather(c + 1, (c + 1) % 2)
            gathers[c].wait()
            writes[c] = pltpu.async_copy(
                rows_v.at[b], out_hbm.at[pl.ds(base + c * ch, ch)],
                wsem.at[b])
        writes[n_chunks - 2].wait()
        writes[n_chunks - 1].wait()

    return gather_k


def _mlp_body(x_ref, wcat_ref, b1_ref, w2_ref, b2_ref, w3_ref, b3_ref, o_ref):
    x = x_ref[...]
    y = jnp.dot(x, wcat_ref[...], preferred_element_type=jnp.float32)
    h1 = jnp.maximum(y[:, :-1] + b1_ref[...], 0.0)
    s = y[:, -1:]
    h2 = jnp.maximum(
        jnp.dot(h1, w2_ref[...], preferred_element_type=jnp.float32)
        + b2_ref[...], 0.0)
    o_ref[...] = (
        jnp.dot(h2, w3_ref[...], preferred_element_type=jnp.float32)
        + b3_ref[...] + s)


def kernel(fids_batch, emb_w, emb_b, W1, b1, W2, b2, W3, b3):
    B, F = fids_batch.shape
    V, D = emb_w.shape
    H1, IN = W1.shape
    H2 = W2.shape[0]
    RW = 8
    N = B * F

    tab = (jnp.pad(emb_w, ((0, 0), (0, RW - D)))
           + jnp.pad(emb_b[:, None], ((0, 0), (D, RW - D - 1))))
    fids_flat = fids_batch.reshape(N)

    gathered = _make_gather(N, RW, 8)(tab, fids_flat)
    X = gathered.reshape(B, F * RW)

    W1r = W1.T.reshape(F, D, H1)
    W1p = jnp.concatenate(
        [W1r, jnp.zeros((F, RW - D, H1), jnp.float32)],
        axis=1).reshape(F * RW, H1)
    mcol = jnp.tile(
        jnp.array([0.0] * D + [1.0] + [0.0] * (RW - D - 1),
                  jnp.float32), F)[:, None]
    Wcat = jnp.concatenate([W1p, mcol], axis=1)

    BM = 1024
    out2 = pl.pallas_call(
        _mlp_body,
        grid=(B // BM,),
        in_specs=[
            pl.BlockSpec((BM, F * RW), lambda i: (i, 0)),
            pl.BlockSpec((F * RW, H1 + 1), lambda i: (0, 0)),
            pl.BlockSpec((1, H1), lambda i: (0, 0)),
            pl.BlockSpec((H1, H2), lambda i: (0, 0)),
            pl.BlockSpec((1, H2), lambda i: (0, 0)),
            pl.BlockSpec((H2, 1), lambda i: (0, 0)),
            pl.BlockSpec((1, 1), lambda i: (0, 0)),
        ],
        out_specs=pl.BlockSpec((BM, 1), lambda i: (i, 0)),
        out_shape=jax.ShapeDtypeStruct((B, 1), jnp.float32),
    )(X, Wcat, b1[None, :], W2.T, b2[None, :], W3.T, b3[None, :])

    return lax.squeeze(out2, (1,))


# speedup vs baseline: 1.3779x; 1.3779x over previous
import functools

import jax
import jax.numpy as jnp
from jax import lax
from jax.experimental import pallas as pl
from jax.experimental.pallas import tpu as pltpu
from jax.experimental.pallas import tpu_sc as plsc

_NC = 2
_NS = 16


@functools.lru_cache(maxsize=None)
def _make_gather(n_idx, row_w, n_chunks):
    nw = _NC * _NS
    per_w = n_idx // nw
    ch = per_w // n_chunks
    assert per_w % n_chunks == 0 and ch % 8 == 0

    mesh = plsc.VectorSubcoreMesh(
        core_axis_name="c", subcore_axis_name="s",
        num_cores=_NC, num_subcores=_NS)

    @functools.partial(
        pl.kernel,
        out_type=jax.ShapeDtypeStruct((n_idx, row_w), jnp.float32),
        mesh=mesh,
        scratch_types=[
            pltpu.VMEM((2, ch), jnp.int32),
            pltpu.VMEM((2, ch, row_w), jnp.float32),
            pltpu.SemaphoreType.DMA((2,)),
            pltpu.SemaphoreType.DMA((2,)),
        ],
        compiler_params=pltpu.CompilerParams(use_tc_tiling_on_sc=False),
    )
    def gather_k(tab_hbm, idx_hbm, out_hbm, idx_v, rows_v, gsem, wsem):
        wid = lax.axis_index("s") * _NC + lax.axis_index("c")
        base = wid * per_w

        def start_gather(c, b):
            pltpu.sync_copy(idx_hbm.at[pl.ds(base + c * ch, ch)], idx_v.at[b])
            return pltpu.async_copy(
                tab_hbm.at[idx_v.at[b]], rows_v.at[b], gsem.at[b])

        gathers = {0: start_gather(0, 0)}
        writes = {}
        for c in range(n_chunks):
            b = c % 2
            if c + 1 < n_chunks:
                if c - 1 >= 0:
                    writes[c - 1].wait()
                gathers[c + 1] = start_gather(c + 1, (c + 1) % 2)
            gathers[c].wait()
            writes[c] = pltpu.async_copy(
                rows_v.at[b], out_hbm.at[pl.ds(base + c * ch, ch)],
                wsem.at[b])
        writes[n_chunks - 2].wait()
        writes[n_chunks - 1].wait()

    return gather_k


def _mlp_body(x_ref, wcat_ref, b1_ref, w2_ref, b2_ref, w3_ref, b3_ref, o_ref):
    x = x_ref[...]
    y = jnp.dot(x, wcat_ref[...], preferred_element_type=jnp.float32)
    h1 = jnp.maximum(y[:, :-1] + b1_ref[...], 0.0)
    s = y[:, -1:]
    h2 = jnp.maximum(
        jnp.dot(h1, w2_ref[...], preferred_element_type=jnp.float32)
        + b2_ref[...], 0.0)
    o_ref[...] = (
        jnp.dot(h2, w3_ref[...], preferred_element_type=jnp.float32)
        + b3_ref[...] + s)


def kernel(fids_batch, emb_w, emb_b, W1, b1, W2, b2, W3, b3):
    B, F = fids_batch.shape
    V, D = emb_w.shape
    H1, IN = W1.shape
    H2 = W2.shape[0]
    RW = 8
    N = B * F

    tab = jnp.concatenate(
        [emb_w, emb_b[:, None], jnp.zeros((V, RW - D - 1), jnp.float32)],
        axis=1)
    fids_flat = fids_batch.reshape(N)

    gathered = _make_gather(N, RW, 8)(tab, fids_flat)
    X = gathered.reshape(B, F * RW)

    W1r = W1.T.reshape(F, D, H1)
    W1p = jnp.concatenate(
        [W1r, jnp.zeros((F, RW - D, H1), jnp.float32)],
        axis=1).reshape(F * RW, H1)
    mcol = jnp.tile(
        jnp.array([0.0] * D + [1.0] + [0.0] * (RW - D - 1),
                  jnp.float32), F)[:, None]
    Wcat = jnp.concatenate([W1p, mcol], axis=1)

    BM = 1024
    out2 = pl.pallas_call(
        _mlp_body,
        grid=(B // BM,),
        in_specs=[
            pl.BlockSpec((BM, F * RW), lambda i: (i, 0)),
            pl.BlockSpec((F * RW, H1 + 1), lambda i: (0, 0)),
            pl.BlockSpec((1, H1), lambda i: (0, 0)),
            pl.BlockSpec((H1, H2), lambda i: (0, 0)),
            pl.BlockSpec((1, H2), lambda i: (0, 0)),
            pl.BlockSpec((H2, 1), lambda i: (0, 0)),
            pl.BlockSpec((1, 1), lambda i: (0, 0)),
        ],
        out_specs=pl.BlockSpec((BM, 1), lambda i: (i, 0)),
        out_shape=jax.ShapeDtypeStruct((B, 1), jnp.float32),
    )(X, Wcat, b1[None, :], W2.T, b2[None, :], W3.T, b3[None, :])

    return lax.squeeze(out2, (1,))
